# baseline (speedup 1.0000x reference)
import functools

import jax
import jax.numpy as jnp
from jax import lax
from jax.experimental import pallas as pl
from jax.experimental.pallas import tpu as pltpu
from jax.experimental.pallas import tpu_sc as plsc

NA, NK, NM, ND, NG_ = 3, 8, 32, 32, 12
NAK = NA * NK
BLOCK = 128
MERGED = BLOCK * ND
NQ_ = 1024
NROWS = NAK * ND
NW = 32
RPW = NROWS // NW
NCHUNK = 2
H = NQ_ // NCHUNK


def _sc_select_body(i0, qT_hbm, QT_hbm, gB_hbm, e5_hbm, esel_hbm,
                    q_v, Qv_v, gB_v, M_v, plane_v, rows8_v):
    wid = lax.axis_index("s") * 2 + lax.axis_index("c")

    pltpu.sync_copy(qT_hbm, q_v)
    pltpu.sync_copy(QT_hbm, Qv_v)
    pltpu.sync_copy(gB_hbm, gB_v)

    one16 = jnp.ones((16,), jnp.float32)
    zero16 = jnp.zeros((16,), jnp.float32)

    def _match(blk, carry):
        ibase = blk * 16
        qc = [q_v[c, pl.ds(ibase, 16)] for c in range(3)]
        Qc = [Qv_v[c, pl.ds(ibase, 16)] for c in range(3)]
        for j in range(NG_):
            cond = None
            for c in range(3):
                gcj = gB_v[c * NG_ + j]
                unf = qc[c] + gcj
                ok = jnp.abs(Qc[c] - unf) <= 1e-5 + 1e-5 * jnp.abs(unf)
                cond = ok if cond is None else jnp.logical_and(cond, ok)
            M_v[j, pl.ds(ibase, 16)] = jnp.where(cond, one16, zero16)
        return carry

    lax.fori_loop(0, H // 16, _match, 0)

    for tb in range(RPW // 8):
        for t8 in range(8):
            r = wid * RPW + tb * 8 + t8
            ak = r // ND
            d = r % ND
            pltpu.sync_copy(e5_hbm.at[ak, d, :, pl.ds(i0, H)], plane_v)

            def _reduce(blk, carry):
                sl = pl.ds(blk * 16, 16)
                acc = plane_v[0, sl] * M_v[0, sl]
                for j in range(1, NG_):
                    acc = acc + plane_v[j, sl] * M_v[j, sl]
                rows8_v[t8, sl] = acc
                return carry

            lax.fori_loop(0, H // 16, _reduce, 0)
        row0 = pl.multiple_of(wid * RPW + tb * 8, 8)
        pltpu.sync_copy(rows8_v, esel_hbm.at[pl.ds(row0, 8), :])


def _make_sc_select(i0):
    @functools.partial(
        pl.kernel,
        out_type=jax.ShapeDtypeStruct((NROWS, H), jnp.float32),
        mesh=plsc.VectorSubcoreMesh(core_axis_name="c", subcore_axis_name="s"),
        scratch_types=[
            pltpu.VMEM((3, H), jnp.float32),
            pltpu.VMEM((3, H), jnp.float32),
            pltpu.VMEM((3 * NG_, 16), jnp.float32),
            pltpu.VMEM((NG_, H), jnp.float32),
            pltpu.VMEM((NG_, H), jnp.float32),
            pltpu.VMEM((8, H), jnp.float32),
        ],
        name=f"sc_select_{i0}",
    )
    def _sc(qT, QT, gB, e5, esel, *scratch):
        _sc_select_body(i0, qT, QT, gB, e5, esel, *scratch)

    return _sc


_SC_SELECT = [_make_sc_select(c * H) for c in range(NCHUNK)]


def _unfold_kernel(om_r_ref, om_c_ref, S_ref, R_ref, et_ref, esel_ref,
                   out_ref, P_scr):
    a = pl.program_id(1)

    @pl.when(a == 0)
    def _init():
        P_scr[...] = jnp.zeros_like(P_scr)

    for k in range(NK):
        es = esel_ref[0, k]
        W = jnp.dot(R_ref[...], es, preferred_element_type=jnp.float32)
        B = W * S_ref[...]
        t_k = et_ref[0, k]
        dots = jnp.dot(t_k, B, preferred_element_type=jnp.float32)
        P_scr[...] = P_scr[...] + dots * dots

    @pl.when(a == NA - 1)
    def _finish():
        eq = (om_r_ref[...] == om_c_ref[...]).astype(jnp.float32)
        out_ref[...] = jnp.dot(
            P_scr[...].T, eq, preferred_element_type=jnp.float32
        ) * (4.0 / 12.0)


def _tc_chunk(c, om_r, om_c, S, R, et2, esel4):
    boff = c * (H // BLOCK)
    grid = (H // BLOCK, NA)
    return pl.pallas_call(
        _unfold_kernel,
        grid=grid,
        in_specs=[
            pl.BlockSpec((NM, 1), lambda b, a: (0, 0)),
            pl.BlockSpec((1, NM), lambda b, a: (0, 0)),
            pl.BlockSpec((MERGED, BLOCK), lambda b, a: (0, 0)),
            pl.BlockSpec((MERGED, ND), lambda b, a: (0, 0)),
            pl.BlockSpec((1, NK, NM, MERGED),
                         lambda b, a: (a, 0, 0, b + boff)),
            pl.BlockSpec((1, NK, ND, BLOCK), lambda b, a: (a, 0, 0, b)),
        ],
        out_specs=pl.BlockSpec((BLOCK, NM), lambda b, a: (b, 0)),
        out_shape=jax.ShapeDtypeStruct((H, NM), jnp.float32),
        scratch_shapes=[pltpu.VMEM((NM, BLOCK), jnp.float32)],
        name=f"tc_unfold_{c}",
    )(om_r, om_c, S, R, et2, esel4)


@jax.jit
def kernel(q, Q, omega, e_tilde, e, g, G):
    nq = q.shape[0]
    qT = q.T
    QT = Q.T
    gG = g - G[None, :]
    gB = jnp.broadcast_to(gG.T.reshape(3 * NG_, 1), (3 * NG_, 16))
    om_r = omega.reshape(NM, 1)
    om_c = omega.reshape(1, NM)
    et2 = e_tilde.reshape(NA, NK, NM, nq * ND)
    e5 = e.reshape(NAK, ND, NG_, nq)
    rows = jnp.arange(MERGED, dtype=jnp.int32)
    S = (rows[:, None] // ND
         == jnp.arange(BLOCK, dtype=jnp.int32)[None, :]).astype(jnp.float32)
    R = (rows[:, None] % ND
         == jnp.arange(ND, dtype=jnp.int32)[None, :]).astype(jnp.float32)

    outs = []
    for c in range(NCHUNK):
        sl = lax.slice_in_dim(qT, c * H, (c + 1) * H, axis=1)
        sQ = lax.slice_in_dim(QT, c * H, (c + 1) * H, axis=1)
        esel = _SC_SELECT[c](sl, sQ, gB, e5)
        esel4 = esel.reshape(NA, NK, ND, H)
        outs.append(_tc_chunk(c, om_r, om_c, S, R, et2, esel4))
    return jnp.concatenate(outs, axis=0)


# speedup vs baseline: 1.0134x; 1.0134x over previous
import functools

import jax
import jax.numpy as jnp
from jax import lax
from jax.experimental import pallas as pl
from jax.experimental.pallas import tpu as pltpu
from jax.experimental.pallas import tpu_sc as plsc

NA, NK, NM, ND, NG_ = 3, 8, 32, 32, 12
NAK = NA * NK
BLOCK = 128
MERGED = BLOCK * ND
NQ_ = 1024
NROWS = NAK * ND
NW = 32
RPW = NROWS // NW


def _sc_select_body(qT_hbm, QT_hbm, gB_hbm, e5_hbm, esel_hbm,
                    q_v, Qv_v, gB_v, M_v, plane_v, plane2_v, rows8_v,
                    sem_a, sem_b):
    wid = lax.axis_index("s") * 2 + lax.axis_index("c")

    pltpu.sync_copy(qT_hbm, q_v)
    pltpu.sync_copy(QT_hbm, Qv_v)
    pltpu.sync_copy(gB_hbm, gB_v)

    one16 = jnp.ones((16,), jnp.float32)
    zero16 = jnp.zeros((16,), jnp.float32)

    def _match(blk, carry):
        ibase = blk * 16
        qc = [q_v[c, pl.ds(ibase, 16)] for c in range(3)]
        Qc = [Qv_v[c, pl.ds(ibase, 16)] for c in range(3)]
        for j in range(NG_):
            cond = None
            for c in range(3):
                gcj = gB_v[c * NG_ + j]
                unf = qc[c] + gcj
                ok = jnp.abs(Qc[c] - unf) <= 1e-5 + 1e-5 * jnp.abs(unf)
                cond = ok if cond is None else jnp.logical_and(cond, ok)
            M_v[j, pl.ds(ibase, 16)] = jnp.where(cond, one16, zero16)
        return carry

    lax.fori_loop(0, NQ_ // 16, _match, 0)

    planes = (plane_v, plane2_v)
    sems = (sem_a, sem_b)

    def _plane_src(t):
        r = wid * RPW + t
        return e5_hbm.at[r // ND, r % ND]

    pltpu.async_copy(_plane_src(0), planes[0], sems[0])
    for tb in range(RPW // 8):
        for t8 in range(8):
            t = tb * 8 + t8
            cur = t % 2
            if t + 1 < RPW:
                pltpu.async_copy(_plane_src(t + 1), planes[1 - cur],
                                 sems[1 - cur])
            pltpu.make_async_copy(_plane_src(t), planes[cur], sems[cur]).wait()
            cur_plane = planes[cur]

            def _reduce(blk, carry):
                sl = pl.ds(blk * 16, 16)
                acc = cur_plane[0, sl] * M_v[0, sl]
                for j in range(1, NG_):
                    acc = acc + cur_plane[j, sl] * M_v[j, sl]
                rows8_v[t8, sl] = acc
                return carry

            lax.fori_loop(0, NQ_ // 16, _reduce, 0)
        row0 = pl.multiple_of(wid * RPW + tb * 8, 8)
        pltpu.sync_copy(rows8_v, esel_hbm.at[pl.ds(row0, 8), :])


@functools.partial(
    pl.kernel,
    out_type=jax.ShapeDtypeStruct((NROWS, NQ_), jnp.float32),
    mesh=plsc.VectorSubcoreMesh(core_axis_name="c", subcore_axis_name="s"),
    scratch_types=[
        pltpu.VMEM((3, NQ_), jnp.float32),
        pltpu.VMEM((3, NQ_), jnp.float32),
        pltpu.VMEM((3 * NG_, 16), jnp.float32),
        pltpu.VMEM((NG_, NQ_), jnp.float32),
        pltpu.VMEM((NG_, NQ_), jnp.float32),
        pltpu.VMEM((NG_, NQ_), jnp.float32),
        pltpu.VMEM((8, NQ_), jnp.float32),
        pltpu.SemaphoreType.DMA,
        pltpu.SemaphoreType.DMA,
    ],
)
def _sc_select(qT, QT, gB, e5, esel, *scratch):
    _sc_select_body(qT, QT, gB, e5, esel, *scratch)


def _unfold_kernel(om_r_ref, om_c_ref, S_ref, R_ref, et_ref, esel_ref,
                   out_ref, P_scr):
    a = pl.program_id(1)

    @pl.when(a == 0)
    def _init():
        P_scr[...] = jnp.zeros_like(P_scr)

    for k in range(NK):
        es = esel_ref[0, k]
        W = jnp.dot(R_ref[...], es, preferred_element_type=jnp.float32)
        B = W * S_ref[...]
        t_k = et_ref[0, k]
        dots = jnp.dot(t_k, B, preferred_element_type=jnp.float32)
        P_scr[...] = P_scr[...] + dots * dots

    @pl.when(a == NA - 1)
    def _finish():
        eq = (om_r_ref[...] == om_c_ref[...]).astype(jnp.float32)
        out_ref[...] = jnp.dot(
            P_scr[...].T, eq, preferred_element_type=jnp.float32
        ) * (4.0 / 12.0)


@jax.jit
def kernel(q, Q, omega, e_tilde, e, g, G):
    nq = q.shape[0]
    qT = q.T
    QT = Q.T
    gG = g - G[None, :]
    gB = jnp.broadcast_to(gG.T.reshape(3 * NG_, 1), (3 * NG_, 16))
    om_r = omega.reshape(NM, 1)
    om_c = omega.reshape(1, NM)
    et2 = e_tilde.reshape(NA, NK, NM, nq * ND)
    e5 = e.reshape(NAK, ND, NG_, nq)
    rows = jnp.arange(MERGED, dtype=jnp.int32)
    S = (rows[:, None] // ND
         == jnp.arange(BLOCK, dtype=jnp.int32)[None, :]).astype(jnp.float32)
    R = (rows[:, None] % ND
         == jnp.arange(ND, dtype=jnp.int32)[None, :]).astype(jnp.float32)

    esel = _sc_select(qT, QT, gB, e5)
    esel4 = esel.reshape(NA, NK, ND, nq)

    grid = (nq // BLOCK, NA)
    out = pl.pallas_call(
        _unfold_kernel,
        grid=grid,
        in_specs=[
            pl.BlockSpec((NM, 1), lambda b, a: (0, 0)),
            pl.BlockSpec((1, NM), lambda b, a: (0, 0)),
            pl.BlockSpec((MERGED, BLOCK), lambda b, a: (0, 0)),
            pl.BlockSpec((MERGED, ND), lambda b, a: (0, 0)),
            pl.BlockSpec((1, NK, NM, MERGED), lambda b, a: (a, 0, 0, b)),
            pl.BlockSpec((1, NK, ND, BLOCK), lambda b, a: (a, 0, 0, b)),
        ],
        out_specs=pl.BlockSpec((BLOCK, NM), lambda b, a: (b, 0)),
        out_shape=jax.ShapeDtypeStruct((nq, NM), jnp.float32),
        scratch_shapes=[pltpu.VMEM((NM, BLOCK), jnp.float32)],
    )(om_r, om_c, S, R, et2, esel4)
    return out


# speedup vs baseline: 1.0228x; 1.0093x over previous
import functools

import jax
import jax.numpy as jnp
from jax import lax
from jax.experimental import pallas as pl
from jax.experimental.pallas import tpu as pltpu
from jax.experimental.pallas import tpu_sc as plsc

NA, NK, NM, ND, NG_ = 3, 8, 32, 32, 12
NAK = NA * NK
BLOCK = 128
MERGED = BLOCK * ND
NQ_ = 1024
NROWS = NAK * ND
NW = 32
RPW = NROWS // NW


def _sc_select_body(qT_hbm, QT_hbm, gB_hbm, e5_hbm, esel_hbm,
                    q_v, Qv_v, gB_v, M_v, plane_v, rows8_v):
    wid = lax.axis_index("s") * 2 + lax.axis_index("c")

    pltpu.sync_copy(qT_hbm, q_v)
    pltpu.sync_copy(QT_hbm, Qv_v)
    pltpu.sync_copy(gB_hbm, gB_v)

    one16 = jnp.ones((16,), jnp.float32)
    zero16 = jnp.zeros((16,), jnp.float32)

    def _match(blk, carry):
        ibase = blk * 16
        qc = [q_v[c, pl.ds(ibase, 16)] for c in range(3)]
        Qc = [Qv_v[c, pl.ds(ibase, 16)] for c in range(3)]
        for j in range(NG_):
            cond = None
            for c in range(3):
                gcj = gB_v[c * NG_ + j]
                unf = qc[c] + gcj
                ok = jnp.abs(Qc[c] - unf) <= 1e-5 + 1e-5 * jnp.abs(unf)
                cond = ok if cond is None else jnp.logical_and(cond, ok)
            M_v[j, pl.ds(ibase, 16)] = jnp.where(cond, one16, zero16)
        return carry

    lax.fori_loop(0, NQ_ // 16, _match, 0)

    for tb in range(RPW // 8):
        for t8 in range(8):
            r = wid * RPW + tb * 8 + t8
            ak = r // ND
            d = r % ND
            pltpu.sync_copy(e5_hbm.at[ak, d], plane_v)

            def _reduce(blk, carry):
                sl = pl.ds(blk * 16, 16)
                acc = plane_v[0, sl] * M_v[0, sl]
                for j in range(1, NG_):
                    acc = acc + plane_v[j, sl] * M_v[j, sl]
                rows8_v[t8, sl] = acc
                return carry

            lax.fori_loop(0, NQ_ // 16, _reduce, 0)
        row0 = pl.multiple_of(wid * RPW + tb * 8, 8)
        pltpu.sync_copy(rows8_v, esel_hbm.at[pl.ds(row0, 8), :])


@functools.partial(
    pl.kernel,
    out_type=jax.ShapeDtypeStruct((NROWS, NQ_), jnp.float32),
    mesh=plsc.VectorSubcoreMesh(core_axis_name="c", subcore_axis_name="s"),
    scratch_types=[
        pltpu.VMEM((3, NQ_), jnp.float32),
        pltpu.VMEM((3, NQ_), jnp.float32),
        pltpu.VMEM((3 * NG_, 16), jnp.float32),
        pltpu.VMEM((NG_, NQ_), jnp.float32),
        pltpu.VMEM((NG_, NQ_), jnp.float32),
        pltpu.VMEM((8, NQ_), jnp.float32),
    ],
)
def _sc_select(qT, QT, gB, e5, esel, *scratch):
    _sc_select_body(qT, QT, gB, e5, esel, *scratch)


def _unfold_kernel(om_r_ref, om_c_ref, S_ref, R_ref, et_ref, esel_ref,
                   out_ref, P_scr):
    a = pl.program_id(1)

    @pl.when(a == 0)
    def _init():
        P_scr[...] = jnp.zeros_like(P_scr)

    for k in range(NK):
        es = esel_ref[0, k]
        W = jnp.dot(R_ref[...], es, preferred_element_type=jnp.float32)
        B = W * S_ref[...]
        t_k = et_ref[0, k]
        dots = jnp.dot(t_k, B, preferred_element_type=jnp.float32)
        P_scr[...] = P_scr[...] + dots * dots

    @pl.when(a == NA - 1)
    def _finish():
        eq = (om_r_ref[...] == om_c_ref[...]).astype(jnp.float32)
        out_ref[...] = jnp.dot(
            P_scr[...].T, eq, preferred_element_type=jnp.float32
        ) * (4.0 / 12.0)


@jax.jit
def kernel(q, Q, omega, e_tilde, e, g, G):
    nq = q.shape[0]
    qT = q.T
    QT = Q.T
    gG = g - G[None, :]
    gB = jnp.broadcast_to(gG.T.reshape(3 * NG_, 1), (3 * NG_, 16))
    om_r = omega.reshape(NM, 1)
    om_c = omega.reshape(1, NM)
    et2 = e_tilde.reshape(NA, NK, NM, nq * ND)
    e5 = e.reshape(NAK, ND, NG_, nq)
    rows = jnp.arange(MERGED, dtype=jnp.int32)
    S = (rows[:, None] // ND
         == jnp.arange(BLOCK, dtype=jnp.int32)[None, :]).astype(jnp.float32)
    R = (rows[:, None] % ND
         == jnp.arange(ND, dtype=jnp.int32)[None, :]).astype(jnp.float32)

    esel = _sc_select(qT, QT, gB, e5)
    esel4 = esel.reshape(NA, NK, ND, nq)

    grid = (nq // BLOCK, NA)
    out = pl.pallas_call(
        _unfold_kernel,
        grid=grid,
        in_specs=[
            pl.BlockSpec((NM, 1), lambda b, a: (0, 0)),
            pl.BlockSpec((1, NM), lambda b, a: (0, 0)),
            pl.BlockSpec((MERGED, BLOCK), lambda b, a: (0, 0)),
            pl.BlockSpec((MERGED, ND), lambda b, a: (0, 0)),
            pl.BlockSpec((1, NK, NM, MERGED), lambda b, a: (a, 0, 0, b)),
            pl.BlockSpec((1, NK, ND, BLOCK), lambda b, a: (a, 0, 0, b)),
        ],
        out_specs=pl.BlockSpec((BLOCK, NM), lambda b, a: (b, 0)),
        out_shape=jax.ShapeDtypeStruct((nq, NM), jnp.float32),
        scratch_shapes=[pltpu.VMEM((NM, BLOCK), jnp.float32)],
    )(om_r, om_c, S, R, et2, esel4)
    return out
